# baseline (speedup 1.0000x reference)
import functools

import jax
import jax.numpy as jnp
from jax import lax
from jax.experimental import pallas as pl
from jax.experimental.pallas import tpu as pltpu
from jax.experimental.pallas import tpu_sc as plsc

_CP = pltpu.CompilerParams(vmem_limit_bytes=67000000)

B, N, KD, VD = 4096, 200, 64, 64
BL = 128
_NW = 32
_ROWS_PER_W = (N + 1 + _NW - 1) // _NW


def _tc_body(nk_ref, nv_ref, km_ref, vm_ref, gate_ref, it_ref, wb_ref,
             out_vm_ref, out_read_ref):
    km = km_ref[...]
    vm = vm_ref[...]
    nv = nv_ref[...]
    it = it_ref[...]
    w = wb_ref[0, 0]
    bconf = wb_ref[0, 1]

    slot = jax.lax.broadcasted_iota(jnp.int32, (N, BL), 0)
    mask = slot <= it

    sim = jnp.sum(vm * nv[None, :, :], axis=1)
    sim = jnp.where(mask, sim, 0.0)

    m = jnp.max(sim, axis=0, keepdims=True)
    e = jnp.exp(sim - m)
    wv = e / jnp.sum(e, axis=0, keepdims=True)

    conf = jax.nn.sigmoid(sim * w + bconf)

    read_k = jnp.sum(wv[:, None, :] * km, axis=0)
    read_c = jnp.sum(wv * conf, axis=0, keepdims=True)

    scale = jax.nn.sigmoid(gate_ref[...])
    scale = scale * (it > 1).astype(jnp.float32)
    out_read_ref[:KD, :] = read_k * scale
    out_read_ref[KD:, :] = read_c * scale

    out_vm_ref[:N, :, :] = vm
    out_vm_ref[N:, :, :] = nv[None, :, :]


_sc_mesh = plsc.VectorSubcoreMesh(core_axis_name="c", subcore_axis_name="s")


@functools.partial(
    pl.kernel,
    mesh=_sc_mesh,
    out_type=jax.ShapeDtypeStruct((N + 1, KD, B), jnp.float32),
)
def _sc_append(km_hbm, nk_hbm, out_hbm):
    wid = lax.axis_index("s") * 2 + lax.axis_index("c")
    for i in range(_ROWS_PER_W):
        n = wid * _ROWS_PER_W + i
        if _ROWS_PER_W * _NW > N + 1:
            @pl.when(n < N)
            def _():
                pltpu.sync_copy(km_hbm.at[n], out_hbm.at[n])
        else:
            pltpu.sync_copy(km_hbm.at[n], out_hbm.at[n])
    @pl.when(wid == _NW - 1)
    def _():
        pltpu.sync_copy(nk_hbm, out_hbm.at[N])


def kernel(new_key, new_value, key_memory, value_memory, gate, iteration, W_conf, b_conf):
    wb = jnp.concatenate([W_conf[0], b_conf]).reshape(1, 2)
    nkT = new_key.T
    nvT = new_value.T
    kmT = jnp.transpose(key_memory, (1, 2, 0))
    vmT = jnp.transpose(value_memory, (1, 2, 0))
    gateT = gate.T
    itT = iteration.T

    out_km = _sc_append(kmT, nkT)

    grid = (B // BL,)
    col = lambda i: (0, i)
    col3 = lambda i: (0, 0, i)
    fixed = lambda i: (0, 0)
    out_vm, out_read = pl.pallas_call(
        _tc_body,
        grid=grid,
        compiler_params=_CP,
        in_specs=[
            pl.BlockSpec((KD, BL), col),
            pl.BlockSpec((VD, BL), col),
            pl.BlockSpec((N, KD, BL), col3),
            pl.BlockSpec((N, VD, BL), col3),
            pl.BlockSpec((1, BL), col),
            pl.BlockSpec((1, BL), col),
            pl.BlockSpec((1, 2), fixed),
        ],
        out_specs=[
            pl.BlockSpec((N + 1, VD, BL), col3),
            pl.BlockSpec((KD + 1, BL), col),
        ],
        out_shape=[
            jax.ShapeDtypeStruct((N + 1, VD, B), jnp.float32),
            jax.ShapeDtypeStruct((KD + 1, B), jnp.float32),
        ],
    )(nkT, nvT, kmT, vmT, gateT, itT, wb)

    return (jnp.transpose(out_km, (2, 0, 1)),
            jnp.transpose(out_vm, (2, 0, 1)),
            out_read.T)


# speedup vs baseline: 18.2074x; 18.2074x over previous
import functools

import jax
import jax.numpy as jnp
from jax import lax
from jax.experimental import pallas as pl
from jax.experimental.pallas import tpu as pltpu
from jax.experimental.pallas import tpu_sc as plsc

_CP = pltpu.CompilerParams(vmem_limit_bytes=67000000)

B, N, KD, VD = 4096, 200, 64, 64
BL = 128
_NW = 32
_ROWS_PER_W = (N + 1 + _NW - 1) // _NW


def _tc_body(nk_ref, nv_ref, km_ref, vm_ref, gate_ref, it_ref, wb_ref,
             out_vm_ref, out_read_ref):
    km = km_ref[...]
    vm = vm_ref[...]
    nv = nv_ref[...]
    it = it_ref[...]
    w = wb_ref[0, 0]
    bconf = wb_ref[0, 1]

    slot = jax.lax.broadcasted_iota(jnp.int32, (N, BL), 0)
    mask = slot <= it

    sim = jnp.sum(vm * nv[None, :, :], axis=1)
    sim = jnp.where(mask, sim, 0.0)

    m = jnp.max(sim, axis=0, keepdims=True)
    e = jnp.exp(sim - m)
    wv = e / jnp.sum(e, axis=0, keepdims=True)

    conf = jax.nn.sigmoid(sim * w + bconf)

    read_k = jnp.sum(wv[:, None, :] * km, axis=0)
    read_c = jnp.sum(wv * conf, axis=0, keepdims=True)

    scale = jax.nn.sigmoid(gate_ref[...])
    scale = scale * (it > 1).astype(jnp.float32)
    out_read_ref[:KD, :] = read_k * scale
    out_read_ref[KD:, :] = read_c * scale

    out_vm_ref[:N, :, :] = vm
    out_vm_ref[N:, :, :] = nv[None, :, :]


_sc_mesh = plsc.VectorSubcoreMesh(core_axis_name="c", subcore_axis_name="s")


_CH = 8
_NCH = KD // _CH


@functools.partial(
    pl.kernel,
    mesh=_sc_mesh,
    out_type=jax.ShapeDtypeStruct((N + 1, KD, B), jnp.float32),
    scratch_types=[
        pltpu.VMEM((_CH, B), jnp.float32),
        pltpu.VMEM((_CH, B), jnp.float32),
        pltpu.SemaphoreType.DMA,
        pltpu.SemaphoreType.DMA,
        pltpu.SemaphoreType.DMA,
        pltpu.SemaphoreType.DMA,
    ],
)
def _sc_append(km_hbm, nk_hbm, out_hbm, b0, b1, si0, si1, so0, so1):
    wid = lax.axis_index("s") * 2 + lax.axis_index("c")
    bufs = (b0, b1)
    isems = (si0, si1)
    osems = (so0, so1)

    def copy_row(src_row, dst_row):
        outs = [None, None]
        for j in range(_NCH):
            k = j % 2
            if outs[k] is not None:
                outs[k].wait()
            sl = pl.ds(j * _CH, _CH)
            pltpu.async_copy(src_row.at[sl], bufs[k], isems[k]).wait()
            outs[k] = pltpu.async_copy(bufs[k], dst_row.at[sl], osems[k])
        for k in range(2):
            if outs[k] is not None:
                outs[k].wait()

    for i in range(_ROWS_PER_W):
        n = wid + _NW * i
        @pl.when(n < N)
        def _():
            copy_row(km_hbm.at[n], out_hbm.at[n])
    @pl.when(wid == _NW - 1)
    def _():
        copy_row(nk_hbm, out_hbm.at[N])


def kernel(new_key, new_value, key_memory, value_memory, gate, iteration, W_conf, b_conf):
    wb = jnp.concatenate([W_conf[0], b_conf]).reshape(1, 2)
    nkT = new_key.T
    nvT = new_value.T
    kmT = jnp.transpose(key_memory, (1, 2, 0))
    vmT = jnp.transpose(value_memory, (1, 2, 0))
    gateT = gate.T
    itT = iteration.T

    out_km = _sc_append(kmT, nkT)

    grid = (B // BL,)
    col = lambda i: (0, i)
    col3 = lambda i: (0, 0, i)
    fixed = lambda i: (0, 0)
    out_vm, out_read = pl.pallas_call(
        _tc_body,
        grid=grid,
        compiler_params=_CP,
        in_specs=[
            pl.BlockSpec((KD, BL), col),
            pl.BlockSpec((VD, BL), col),
            pl.BlockSpec((N, KD, BL), col3),
            pl.BlockSpec((N, VD, BL), col3),
            pl.BlockSpec((1, BL), col),
            pl.BlockSpec((1, BL), col),
            pl.BlockSpec((1, 2), fixed),
        ],
        out_specs=[
            pl.BlockSpec((N + 1, VD, BL), col3),
            pl.BlockSpec((KD + 1, BL), col),
        ],
        out_shape=[
            jax.ShapeDtypeStruct((N + 1, VD, B), jnp.float32),
            jax.ShapeDtypeStruct((KD + 1, B), jnp.float32),
        ],
    )(nkT, nvT, kmT, vmT, gateT, itT, wb)

    return (jnp.transpose(out_km, (2, 0, 1)),
            jnp.transpose(out_vm, (2, 0, 1)),
            out_read.T)
